# Initial kernel scaffold; baseline (speedup 1.0000x reference)
import jax
import jax.numpy as jnp
from jax.experimental import pallas as pl


def kernel(x, gate_w, gate_b, expert_w, expert_b):
    raise NotImplementedError("write your pallas kernel here")


# speedup vs baseline: 1.1355x; 1.1355x over previous
import functools

import jax
import jax.numpy as jnp
from jax.experimental import pallas as pl
from jax.experimental.pallas import tpu as pltpu

N, D, E, TOP_K = 8192, 2048, 8, 2
BN = 512


def _moe_body(x_ref, gw_ref, gb_ref, w_ref, b_ref, out_ref, wmask_ref):
    e = pl.program_id(1)

    @pl.when(e == 0)
    def _gate():
        x = x_ref[...]
        logits = jax.lax.dot_general(
            x, gw_ref[...], (((1,), (1,)), ((), ())),
            preferred_element_type=jnp.float32) + gb_ref[...]
        probs = jax.nn.softmax(logits, axis=-1)
        cols = jax.lax.broadcasted_iota(jnp.int32, probs.shape, 1)
        i1 = jnp.argmax(probs, axis=-1, keepdims=True)
        p_masked = jnp.where(cols == i1, -jnp.inf, probs)
        i2 = jnp.argmax(p_masked, axis=-1, keepdims=True)
        sel = (cols == i1) | (cols == i2)
        wmask_ref[...] = jnp.where(sel, probs, 0.0)

    x = x_ref[...]
    w = w_ref[0]
    y = jax.lax.dot_general(x, w, (((1,), (1,)), ((), ())),
                            preferred_element_type=jnp.float32)
    wm = wmask_ref[...]
    ecols = jax.lax.broadcasted_iota(jnp.int32, wm.shape, 1)
    pe = jnp.sum(jnp.where(ecols == e, wm, 0.0), axis=-1, keepdims=True)
    y = (y + b_ref[0]) * pe

    @pl.when(e == 0)
    def _init():
        out_ref[...] = y

    @pl.when(e != 0)
    def _acc():
        out_ref[...] += y


@jax.jit
def kernel(x, gate_w, gate_b, expert_w, expert_b):
    gb2 = gate_b.reshape(1, E)
    eb3 = expert_b.reshape(E, 1, D)
    grid = (N // BN, E)
    return pl.pallas_call(
        _moe_body,
        grid=grid,
        in_specs=[
            pl.BlockSpec((BN, D), lambda n, e: (n, 0)),
            pl.BlockSpec((E, D), lambda n, e: (0, 0)),
            pl.BlockSpec((1, E), lambda n, e: (0, 0)),
            pl.BlockSpec((1, D, D), lambda n, e: (e, 0, 0)),
            pl.BlockSpec((1, 1, D), lambda n, e: (e, 0, 0)),
        ],
        out_specs=pl.BlockSpec((BN, D), lambda n, e: (n, 0)),
        out_shape=jax.ShapeDtypeStruct((N, D), jnp.float32),
        scratch_shapes=[pltpu.VMEM((BN, E), jnp.float32)],
    )(x, gate_w, gb2, expert_w, eb3)
